# Initial kernel scaffold; baseline (speedup 1.0000x reference)
import jax
import jax.numpy as jnp
from jax.experimental import pallas as pl


def kernel(predictions, targets):
    raise NotImplementedError("write your pallas kernel here")


# speedup vs baseline: 1.0480x; 1.0480x over previous
import functools

import jax
import jax.numpy as jnp
from jax.experimental import pallas as pl
from jax.experimental.pallas import tpu as pltpu

_S, _B, _C = 7, 2, 20
_D = _B * 5 + _C
_CELLS = _S * _S
_LAMBDA_COORD, _LAMBDA_NOOBJ = 5.0, 0.5


def _loss_body(p_ref, t_ref, o_ref):
    x = p_ref[...]
    y = t_ref[...]
    f32 = x.dtype

    pw0 = x[:, 2]
    ph0 = x[:, 3]
    pw1 = x[:, 7]
    ph1 = x[:, 8]
    tw = y[:, 2]
    th = y[:, 3]
    tconf = y[:, 4]

    i0 = jnp.minimum(pw0, tw) * jnp.minimum(ph0, th)
    i1 = jnp.minimum(pw1, tw) * jnp.minimum(ph1, th)
    ta = tw * th
    u0 = pw0 * ph0 + ta - i0 + 1e-6
    u1 = pw1 * ph1 + ta - i1 + 1e-6
    s = (i1 * u0 > i0 * u1).astype(f32)

    obj = tconf
    nobj = 1.0 - tconf

    d0 = (x[:, 0] - y[:, 0]) ** 2
    d1 = (x[:, 5] - y[:, 0]) ** 2
    for c in range(1, 4):
        d0 += (x[:, c] - y[:, c]) ** 2
        d1 += (x[:, 5 + c] - y[:, c]) ** 2
    coord = jnp.sum(obj * (d0 + s * (d1 - d0)))

    sq01 = (x[:, 4] - tconf) ** 2 + (x[:, 9] - tconf) ** 2
    objl = jnp.sum(obj * sq01)
    nobjl = jnp.sum(nobj * sq01)

    logp = jnp.maximum(jnp.log(x), -100.0)
    log1mp = jnp.maximum(jnp.log1p(-x), -100.0)
    bce = -(y * logp + (1.0 - y) * log1mp)
    ch = jax.lax.broadcasted_iota(jnp.int32, (1, _D, 1), 1)
    cls_cell = jnp.sum(jnp.where(ch >= _B * 5, bce, 0.0), axis=1)
    classl = jnp.sum(obj * cls_cell)

    o_ref[0, :, 0:1] = jnp.reshape(coord, (1, 1))
    o_ref[0, :, 1:2] = jnp.reshape(objl, (1, 1))
    o_ref[0, :, 2:3] = jnp.reshape(nobjl, (1, 1))
    o_ref[0, :, 3:4] = jnp.reshape(classl, (1, 1))


@functools.partial(jax.jit, static_argnames=("bnl",))
def _yolo_loss(predictions, targets, bnl=512):
    n = predictions.shape[0]
    p = jnp.transpose(predictions, (1, 2, 3, 0)).reshape(_CELLS, _D, n)
    t = jnp.transpose(targets, (1, 2, 3, 0)).reshape(_CELLS, _D, n)
    nb = n // bnl
    partial = pl.pallas_call(
        _loss_body,
        grid=(nb,),
        in_specs=[
            pl.BlockSpec((_CELLS, _D, bnl), lambda i: (0, 0, i)),
            pl.BlockSpec((_CELLS, _D, bnl), lambda i: (0, 0, i)),
        ],
        out_specs=pl.BlockSpec((1, 1, 4), lambda i: (i, 0, 0)),
        out_shape=jax.ShapeDtypeStruct((nb, 1, 4), jnp.float32),
        compiler_params=pltpu.CompilerParams(
            dimension_semantics=("parallel",),
        ),
    )(p, t)
    sums = jnp.sum(partial, axis=(0, 1))
    coord, objl, nobjl, classl = sums[0], sums[1], sums[2], sums[3]
    total = (_LAMBDA_COORD * coord + objl + _LAMBDA_NOOBJ * nobjl + classl) / n
    return (total, coord / n, objl / n, nobjl / n, classl / n)


def kernel(predictions, targets):
    return _yolo_loss(predictions, targets)
